# baseline (speedup 1.0000x reference)
import jax
import jax.numpy as jnp
from jax import lax
from jax.experimental import pallas as pl
from jax.experimental.pallas import tpu as pltpu
from jax.experimental.pallas import tpu_sc as plsc

_SIZE = 50000
_EH = 800000
_FINAL_D = 4
_PAD = 50176

_NC = 2
_NS = 16
_NW = _NC * _NS
_IDXB = 128
_NBLKS = _EH // _IDXB
_ZPT = _PAD // _NS

_NPAD = 802816
_SPT = _NPAD // _NW
_SKB = 512
_SNB = _SPT // _SKB
_PASSES = ((0, 0x7FF, 2048), (11, 0x7FF, 2048), (22, 0x3FF, 1024))


def _i32(x):
    return jnp.int32(x)


def _iota16():
    return lax.iota(jnp.int32, 16)


_GDN = lax.GatherDimensionNumbers(
    offset_dims=(), collapsed_slice_dims=(0,), start_index_map=(0,))


def _take(x, idx):
    return lax.gather(x, idx[:, None], _GDN, (1,),
                      mode=lax.GatherScatterMode.PROMISE_IN_BOUNDS)


def _count_body_factory(shift, mask, ndig):
    def body(key_hbm, cnt_hbm, keybuf, hist16, cntv):
        cid = lax.axis_index("c")
        sid = lax.axis_index("s")
        wid = cid * _NS + sid
        iota = _iota16()
        z16 = jnp.zeros((16,), jnp.int32)
        one16 = jnp.ones((16,), jnp.int32)

        def zero(i, c):
            hist16[pl.ds(i * 16, 16)] = z16
            return c

        lax.fori_loop(_i32(0), _i32(ndig), zero, _i32(0))
        base = wid * _SPT

        def blk(b, c):
            off = base + b * _SKB
            pltpu.sync_copy(key_hbm.at[pl.ds(off, _SKB)], keybuf)

            def vec(j, c2):
                k = keybuf[pl.ds(j * 16, 16)]
                d = lax.shift_right_logical(k, _i32(shift)) & _i32(mask)
                plsc.addupdate_scatter(hist16, [d * 16 + iota], one16)
                return c2

            lax.fori_loop(_i32(0), _i32(_SKB // 16), vec, _i32(0))
            return c

        lax.fori_loop(_i32(0), _i32(_SNB), blk, _i32(0))

        def red(i, c):
            acc = z16
            for l in range(16):
                acc = acc + plsc.load_gather(hist16, [i * 256 + iota * 16 + l])
            cntv[pl.ds(i * 16, 16)] = acc
            return c

        lax.fori_loop(_i32(0), _i32(ndig // 16), red, _i32(0))
        pltpu.sync_copy(cntv, cnt_hbm.at[wid])

    return body


def _sc_count(key_pad, shift, mask, ndig):
    f = pl.kernel(
        _count_body_factory(shift, mask, ndig),
        out_type=jax.ShapeDtypeStruct((_NW, ndig), jnp.int32),
        mesh=plsc.VectorSubcoreMesh(
            core_axis_name="c", subcore_axis_name="s",
            num_cores=_NC, num_subcores=_NS),
        compiler_params=pltpu.CompilerParams(needs_layout_passes=False),
        scratch_types=[
            pltpu.VMEM((_SKB,), jnp.int32),
            pltpu.VMEM((ndig * 16,), jnp.int32),
            pltpu.VMEM((ndig,), jnp.int32),
        ],
    )
    return f(key_pad)


def _permute_body_factory(shift, mask, ndig, has_idx):
    def body(*args):
        if has_idx:
            (key_hbm, idx_hbm, cnt_hbm, keyo, idxo,
             cntm, offt, keybuf, idxbuf, dpos, dkey, didx) = args
        else:
            (key_hbm, cnt_hbm, keyo, idxo,
             cntm, offt, keybuf, idxbuf, dpos, dkey, didx) = args
        cid = lax.axis_index("c")
        sid = lax.axis_index("s")
        wid = cid * _NS + sid
        iota = _iota16()
        z16 = jnp.zeros((16,), jnp.int32)
        pltpu.sync_copy(cnt_hbm, cntm)

        def chunk(i, carry):
            def tl(t, tp):
                tot, part = tp
                v = cntm[pl.ds(t * ndig + i * 16, 16)]
                tot = tot + v
                part = part + jnp.where(t < wid, v, z16)
                return (tot, part)

            tot, part = lax.fori_loop(_i32(0), _i32(_NW), tl, (z16, z16))
            incl = plsc.cumsum(tot)
            ex = incl - tot + carry
            offt[pl.ds(i * 16, 16)] = ex + part
            return carry + jnp.sum(tot, dtype=jnp.int32)

        lax.fori_loop(_i32(0), _i32(ndig // 16), chunk, _i32(0))

        base = wid * _SPT

        def blk(b, c):
            off0 = base + b * _SKB
            pltpu.sync_copy(key_hbm.at[pl.ds(off0, _SKB)], keybuf)
            if has_idx:
                pltpu.sync_copy(idx_hbm.at[pl.ds(off0, _SKB)], idxbuf)
            for j in range(_SKB // 16):
                k = keybuf[pl.ds(j * 16, 16)]
                if has_idx:
                    iv = idxbuf[pl.ds(j * 16, 16)]
                else:
                    iv = off0 + _i32(j * 16) + iota
                d = lax.shift_right_logical(k, _i32(shift)) & _i32(mask)
                comb = d * 16 + iota
                sk, sl = plsc.sort_key_val(comb, iota)
                ds_ = lax.shift_right_logical(sk, _i32(4))
                prev = _take(ds_, jnp.maximum(iota - 1, z16))
                isnew = (iota == 0) | (ds_ != prev)
                rs = plsc.cummax(jnp.where(isnew, iota, z16))
                w = iota - rs
                cur = plsc.load_gather(offt, [ds_])
                pos = cur + w
                nxt = _take(ds_, jnp.minimum(iota + 1, jnp.full((16,), 15, jnp.int32)))
                isend = (iota == 15) | (ds_ != nxt)
                plsc.addupdate_scatter(offt, [ds_], w + 1, mask=isend)
                ks = _take(k, sl)
                is_ = _take(iv, sl)
                r0, c0 = j // 8, j % 8
                dpos[r0, pl.ds(c0 * 16, 16)] = pos
                dkey[r0, pl.ds(c0 * 16, 16)] = ks
                didx[r0, pl.ds(c0 * 16, 16)] = is_
            for r in range(_SKB // 128):
                ri = _i32(r)
                pltpu.sync_copy(dkey.at[ri], keyo.at[dpos.at[ri]])
                pltpu.sync_copy(didx.at[ri], idxo.at[dpos.at[ri]])
            return c

        lax.fori_loop(_i32(0), _i32(_SNB), blk, _i32(0))

    return body


def _sc_permute(key_pad, idx_pad, cnt, shift, mask, ndig):
    has_idx = idx_pad is not None
    f = pl.kernel(
        _permute_body_factory(shift, mask, ndig, has_idx),
        out_type=(jax.ShapeDtypeStruct((_NPAD,), jnp.int32),
                  jax.ShapeDtypeStruct((_NPAD,), jnp.int32)),
        mesh=plsc.VectorSubcoreMesh(
            core_axis_name="c", subcore_axis_name="s",
            num_cores=_NC, num_subcores=_NS),
        compiler_params=pltpu.CompilerParams(needs_layout_passes=False),
        scratch_types=[
            pltpu.VMEM((_NW * ndig,), jnp.int32),
            pltpu.VMEM((ndig,), jnp.int32),
            pltpu.VMEM((_SKB,), jnp.int32),
            pltpu.VMEM((_SKB,), jnp.int32),
            pltpu.VMEM((_SKB // 128, 128), jnp.int32),
            pltpu.VMEM((_SKB // 128, 128), jnp.int32),
            pltpu.VMEM((_SKB // 128, 128), jnp.int32),
        ],
    )
    cnt1 = cnt.reshape(-1)
    if has_idx:
        return f(key_pad, idx_pad, cnt1)
    return f(key_pad, cnt1)


def _sc_sort(key_pad):
    k, i = key_pad, None
    for (shift, mask, ndig) in _PASSES:
        cnt = _sc_count(k, shift, mask, ndig)
        k, i = _sc_permute(k, i, cnt, shift, mask, ndig)
    return k, i


_N4 = 4 * _EH
_LR = 2 * _EH
_AKB = 896
_ANB = _SPT // _AKB


def _lane(x, lane):
    iota = _iota16()
    z16 = jnp.zeros((16,), jnp.int32)
    return jnp.sum(jnp.where(iota == lane, x, z16), dtype=jnp.int32)


def _meta_body(key_hbm, meta_hbm, keybuf, pbuf):
    cid = lax.axis_index("c")
    sid = lax.axis_index("s")
    wid = cid * _NS + sid
    iota = _iota16()
    z16 = jnp.zeros((16,), jnp.int32)
    neg16 = jnp.full((16,), -1, jnp.int32)
    base = wid * _SPT
    pbuf[pl.ds(0, 16)] = z16

    @pl.when(wid > 0)
    def _():
        pltpu.sync_copy(key_hbm.at[pl.ds(base - 16, 16)], pbuf)

    pk0 = _lane(pbuf[pl.ds(0, 16)], 15)

    def blk(b, carry):
        pk, mk, mr = carry
        off0 = base + b * _SKB
        pltpu.sync_copy(key_hbm.at[pl.ds(off0, _SKB)], keybuf)
        for j in range(_SKB // 16):
            k = keybuf[pl.ds(j * 16, 16)]
            posv = off0 + _i32(j * 16) + iota
            pv = jnp.where(iota == 0, jnp.broadcast_to(pk, (16,)),
                           _take(k, jnp.maximum(iota - 1, z16)))
            ink = (posv == 0) | (k != pv)
            r = lax.shift_right_logical(k, _i32(16))
            rp = lax.shift_right_logical(pv, _i32(16))
            inr = (posv == 0) | (r != rp)
            mk = jnp.maximum(mk, jnp.max(jnp.where(ink, posv, neg16)))
            mr = jnp.maximum(mr, jnp.max(jnp.where(inr, posv, neg16)))
            pk = _lane(k, 15)
        return (pk, mk, mr)

    pk, mk, mr = lax.fori_loop(
        _i32(0), _i32(_SNB), blk, (pk0, _i32(-1), _i32(-1)))
    outv = jnp.where(iota == 0, jnp.broadcast_to(mk, (16,)),
                     jnp.broadcast_to(mr, (16,)))
    pbuf[pl.ds(0, 16)] = outv
    pltpu.sync_copy(pbuf, meta_hbm.at[wid])


def _sc_meta(key_sorted):
    f = pl.kernel(
        _meta_body,
        out_type=jax.ShapeDtypeStruct((_NW, 16), jnp.int32),
        mesh=plsc.VectorSubcoreMesh(
            core_axis_name="c", subcore_axis_name="s",
            num_cores=_NC, num_subcores=_NS),
        compiler_params=pltpu.CompilerParams(needs_layout_passes=False),
        scratch_types=[
            pltpu.VMEM((_SKB,), jnp.int32),
            pltpu.VMEM((16,), jnp.int32),
        ],
    )
    return f(key_sorted)


def _asm_body(key_hbm, idx_hbm, meta_hbm, cnt_hbm, dsi_hbm, tm0_hbm, tm1_hbm,
              trec_hbm, flr_hbm,
              keybuf, idxbuf, metab, pbuf,
              rfb, rsrb, rb, cb, pb,
              gmm, gcnt, gdr, gdc, gt0, gt1,
              sf0, sf1, sf2, sf3, sp0, sp1, sp2, sp3,
              lv1, lv2, lp1, lp2):
    cid = lax.axis_index("c")
    sid = lax.axis_index("s")
    wid = cid * _NS + sid
    iota = _iota16()
    z16 = jnp.zeros((16,), jnp.int32)
    base = wid * _SPT
    sfs = (sf0, sf1, sf2, sf3)
    sps = (sp0, sp1, sp2, sp3)

    pltpu.sync_copy(meta_hbm, metab)
    pbuf[pl.ds(0, 16)] = z16

    @pl.when(wid > 0)
    def _():
        pltpu.sync_copy(key_hbm.at[pl.ds(base - 16, 16)], pbuf)

    pk0 = _lane(pbuf[pl.ds(0, 16)], 15)
    ck0 = _i32(-1)
    cr0 = _i32(-1)
    for tt in range(_NW):
        mv = metab[pl.ds(tt * 16, 16)]
        use = jnp.int32(tt) < wid
        ck0 = jnp.maximum(ck0, jnp.where(use, _lane(mv, 0), _i32(-1)))
        cr0 = jnp.maximum(cr0, jnp.where(use, _lane(mv, 1), _i32(-1)))

    def blk(b, carry):
        pk, ck, cr = carry
        off0 = base + b * _AKB
        pltpu.sync_copy(key_hbm.at[pl.ds(off0, _AKB)], keybuf)
        pltpu.sync_copy(idx_hbm.at[pl.ds(off0, _AKB)], idxbuf)
        for g in range(_AKB // 128):
            for j in range(8):
                o = g * 128 + j * 16
                k = keybuf[pl.ds(o, 16)]
                iv = idxbuf[pl.ds(o, 16)]
                posv = off0 + _i32(o) + iota
                pv = jnp.where(iota == 0, jnp.broadcast_to(pk, (16,)),
                               _take(k, jnp.maximum(iota - 1, z16)))
                ink = (posv == 0) | (k != pv)
                r = lax.shift_right_logical(k, _i32(16))
                rp = lax.shift_right_logical(pv, _i32(16))
                inr = (posv == 0) | (r != rp)
                rs_k = plsc.cummax(jnp.where(ink, posv, jnp.broadcast_to(ck, (16,))))
                rs_r = plsc.cummax(jnp.where(inr, posv, jnp.broadcast_to(cr, (16,))))
                ck = _lane(rs_k, 15)
                cr = _lane(rs_r, 15)
                pk = _lane(k, 15)
                c = k & _i32(0xFFFF)
                rfb[g, pl.ds(j * 16, 16)] = rs_k
                rsrb[g, pl.ds(j * 16, 16)] = rs_r
                rb[g, pl.ds(j * 16, 16)] = jnp.minimum(r, _i32(_PAD - 1))
                cb[g, pl.ds(j * 16, 16)] = jnp.minimum(c, _i32(_PAD - 1))
                pb[g, pl.ds(j * 16, 16)] = jnp.minimum(iv, _i32(_EH - 1))
        pltpu.sync_copy(idx_hbm.at[rfb], gmm)
        pltpu.sync_copy(cnt_hbm.at[rb], gcnt)
        pltpu.sync_copy(dsi_hbm.at[rb], gdr)
        pltpu.sync_copy(dsi_hbm.at[cb], gdc)
        pltpu.sync_copy(tm0_hbm.at[pb], gt0)
        pltpu.sync_copy(tm1_hbm.at[pb], gt1)
        for g in range(_AKB // 128):
            for j in range(8):
                o = g * 128 + j * 16
                k = keybuf[pl.ds(o, 16)]
                posv = off0 + _i32(o) + iota
                real = posv < _i32(_EH)
                r = lax.shift_right_logical(k, _i32(16))
                c = k & _i32(0xFFFF)
                start = rsrb[g, pl.ds(j * 16, 16)]
                cntv = gcnt[g, pl.ds(j * 16, 16)]
                mm_s = gmm[g, pl.ds(j * 16, 16)]
                dr = gdr[g, pl.ds(j * 16, 16)]
                dc = gdc[g, pl.ds(j * 16, 16)]
                t0 = gt0[g, pl.ds(j * 16, 16)]
                t1 = gt1[g, pl.ds(j * 16, 16)]
                wv = dr * dc
                vb = (plsc.bitcast(t0, jnp.int32), plsc.bitcast(t1, jnp.int32),
                      plsc.bitcast(wv, jnp.int32), plsc.bitcast(-wv, jnp.int32))
                base3 = start * 3 + posv
                dump = _i32(_N4) + iota
                gv = jnp.full((16,), g, jnp.int32)
                ev = _i32(j * 16) + iota
                for fidx in range(4):
                    posf = base3 + _i32(fidx) * cntv
                    dest = jnp.where(real, posf, dump)
                    sps[fidx][g, pl.ds(j * 16, 16)] = dest
                    plsc.store_scatter(
                        sfs[fidx], [gv, ev, z16], r * 4 + _i32(fidx))
                    plsc.store_scatter(
                        sfs[fidx], [gv, ev, z16 + 1], c * 4 + _i32(fidx))
                    plsc.store_scatter(
                        sfs[fidx], [gv, ev, z16 + 2], vb[fidx])
                pbv = pb[g, pl.ds(j * 16, 16)]
                dumplr = _i32(_LR) + iota
                lv1[g, pl.ds(j * 16, 16)] = mm_s + _i32(_EH)
                lp1[g, pl.ds(j * 16, 16)] = jnp.where(real, pbv, dumplr)
                lv2[g, pl.ds(j * 16, 16)] = mm_s
                lp2[g, pl.ds(j * 16, 16)] = jnp.where(real, pbv + _i32(_EH), dumplr)
        for fidx in range(4):
            pltpu.sync_copy(sfs[fidx], trec_hbm.at[sps[fidx]])
        pltpu.sync_copy(lv1, flr_hbm.at[lp1])
        pltpu.sync_copy(lv2, flr_hbm.at[lp2])
        return (pk, ck, cr)

    lax.fori_loop(_i32(0), _i32(_ANB), blk, (pk0, ck0, cr0))


def _sc_asm(key_sorted, idx_sorted, meta, cnt_lo_pad, dsi_pad, tm0, tm1):
    nr = _AKB // 128
    f = pl.kernel(
        _asm_body,
        out_type=(jax.ShapeDtypeStruct((_N4 + 16, 4), jnp.int32),
                  jax.ShapeDtypeStruct((_LR + 16,), jnp.int32)),
        mesh=plsc.VectorSubcoreMesh(
            core_axis_name="c", subcore_axis_name="s",
            num_cores=_NC, num_subcores=_NS),
        compiler_params=pltpu.CompilerParams(needs_layout_passes=False),
        scratch_types=[
            pltpu.VMEM((_AKB,), jnp.int32),
            pltpu.VMEM((_AKB,), jnp.int32),
            pltpu.VMEM((_NW * 16,), jnp.int32),
            pltpu.VMEM((16,), jnp.int32),
            pltpu.VMEM((nr, 128), jnp.int32),
            pltpu.VMEM((nr, 128), jnp.int32),
            pltpu.VMEM((nr, 128), jnp.int32),
            pltpu.VMEM((nr, 128), jnp.int32),
            pltpu.VMEM((nr, 128), jnp.int32),
            pltpu.VMEM((nr, 128), jnp.int32),
            pltpu.VMEM((nr, 128), jnp.int32),
            pltpu.VMEM((nr, 128), jnp.float32),
            pltpu.VMEM((nr, 128), jnp.float32),
            pltpu.VMEM((nr, 128), jnp.float32),
            pltpu.VMEM((nr, 128), jnp.float32),
            pltpu.VMEM((nr, 128, 4), jnp.int32),
            pltpu.VMEM((nr, 128, 4), jnp.int32),
            pltpu.VMEM((nr, 128, 4), jnp.int32),
            pltpu.VMEM((nr, 128, 4), jnp.int32),
            pltpu.VMEM((nr, 128), jnp.int32),
            pltpu.VMEM((nr, 128), jnp.int32),
            pltpu.VMEM((nr, 128), jnp.int32),
            pltpu.VMEM((nr, 128), jnp.int32),
            pltpu.VMEM((nr, 128), jnp.int32),
            pltpu.VMEM((nr, 128), jnp.int32),
            pltpu.VMEM((nr, 128), jnp.int32),
            pltpu.VMEM((nr, 128), jnp.int32),
        ],
    )
    return f(key_sorted, idx_sorted, meta.reshape(-1), cnt_lo_pad, dsi_pad,
             tm0, tm1)


def _iota_body(o_ref):
    o_ref[...] = (lax.broadcasted_iota(jnp.int32, (12544, 128), 0) * 128
                  + lax.broadcasted_iota(jnp.int32, (12544, 128), 1))


def _tc_iota():
    out = pl.pallas_call(
        _iota_body,
        out_shape=jax.ShapeDtypeStruct((12544, 128), jnp.int32),
    )()
    return out.reshape(-1)


def _hist_body(lo_hbm, hi_hbm, cnt_lo_hbm, deg_hbm, key_hbm,
               lo_v, hi_v, key_v, ones_v, zbuf_v, cnt_lo_s, deg_s):
    cid = lax.axis_index("c")
    sid = lax.axis_index("s")
    wid = cid * _NS + sid
    ones16 = jnp.ones((16,), jnp.int32)
    z16 = jnp.zeros((16,), jnp.int32)
    for j in range(_IDXB // 16):
        ones_v[pl.ds(j * 16, 16)] = ones16
    for j in range(_ZPT // 16):
        zbuf_v[pl.ds(j * 16, 16)] = z16
    pltpu.sync_copy(zbuf_v, cnt_lo_s.at[pl.ds(sid * _ZPT, _ZPT)])
    pltpu.sync_copy(zbuf_v, deg_s.at[pl.ds(sid * _ZPT, _ZPT)])
    plsc.subcore_barrier()

    nblk = jnp.int32(_NBLKS // _NW) + jnp.where(
        wid < _NBLKS % _NW, jnp.int32(1), jnp.int32(0))

    def blk(b, carry):
        off = (b * _NW + wid) * _IDXB
        pltpu.sync_copy(lo_hbm.at[pl.ds(off, _IDXB)], lo_v)
        pltpu.sync_copy(hi_hbm.at[pl.ds(off, _IDXB)], hi_v)
        for j in range(_IDXB // 16):
            l16 = lo_v[pl.ds(j * 16, 16)]
            h16 = hi_v[pl.ds(j * 16, 16)]
            key_v[pl.ds(j * 16, 16)] = lax.shift_left(l16, _i32(16)) | h16
        pltpu.sync_copy(key_v, key_hbm.at[pl.ds(off, _IDXB)])
        pltpu.sync_copy(ones_v, cnt_lo_s.at[lo_v], add=True)
        pltpu.sync_copy(ones_v, deg_s.at[lo_v], add=True)
        pltpu.sync_copy(ones_v, deg_s.at[hi_v], add=True)
        return carry

    lax.fori_loop(jnp.int32(0), nblk, blk, jnp.int32(0))

    @pl.when(wid == _NW - 1)
    def _():
        neg16 = jnp.full((16,), -1, jnp.int32)
        for j in range(_IDXB // 16):
            key_v[pl.ds(j * 16, 16)] = neg16

        def pblk(p, c):
            pltpu.sync_copy(key_v, key_hbm.at[pl.ds(_EH + p * _IDXB, _IDXB)])
            return c

        lax.fori_loop(_i32(0), _i32((_NPAD - _EH) // _IDXB), pblk, _i32(0))

    plsc.subcore_barrier()

    @pl.when(sid == 0)
    def _():
        pltpu.sync_copy(cnt_lo_s, cnt_lo_hbm.at[cid])
        pltpu.sync_copy(deg_s, deg_hbm.at[cid])


def _sc_hist(lo32, hi32):
    f = pl.kernel(
        _hist_body,
        out_type=(
            jax.ShapeDtypeStruct((_NC, _PAD), jnp.int32),
            jax.ShapeDtypeStruct((_NC, _PAD), jnp.int32),
            jax.ShapeDtypeStruct((_NPAD,), jnp.int32),
        ),
        mesh=plsc.VectorSubcoreMesh(
            core_axis_name="c", subcore_axis_name="s",
            num_cores=_NC, num_subcores=_NS),
        compiler_params=pltpu.CompilerParams(needs_layout_passes=False),
        scratch_types=[
            pltpu.VMEM((_IDXB,), jnp.int32),
            pltpu.VMEM((_IDXB,), jnp.int32),
            pltpu.VMEM((_IDXB,), jnp.int32),
            pltpu.VMEM((_IDXB,), jnp.int32),
            pltpu.VMEM((_ZPT,), jnp.int32),
            pltpu.VMEM_SHARED((_PAD,), jnp.int32),
            pltpu.VMEM_SHARED((_PAD,), jnp.int32),
        ],
    )
    return f(lo32, hi32)


def _norm_body(cnt_lo_p, deg_p, cnt_lo_ref, deg_ref, dsi_ref, fd_ref):
    cnt = cnt_lo_p[0] + cnt_lo_p[1]
    degi = deg_p[0] + deg_p[1]
    deg = degi.astype(jnp.float32)
    cnt_lo_ref[...] = cnt
    deg_ref[...] = deg
    dsi = lax.rsqrt(deg + 1.0)
    dsi_ref[...] = dsi
    fd_ref[...] = dsi * dsi * deg


def _tc_norm(cnt_lo_part, deg_part):
    return pl.pallas_call(
        _norm_body,
        out_shape=(jax.ShapeDtypeStruct((392, 128), jnp.int32),
                   jax.ShapeDtypeStruct((392, 128), jnp.float32),
                   jax.ShapeDtypeStruct((392, 128), jnp.float32),
                   jax.ShapeDtypeStruct((392, 128), jnp.float32)),
    )(cnt_lo_part.reshape(_NC, 392, 128), deg_part.reshape(_NC, 392, 128))


def kernel(edge_index, diag_maps, tril_maps):
    size, final_d = _SIZE, _FINAL_D
    idt = edge_index.dtype
    row, col = edge_index[0], edge_index[1]
    E = row.shape[0]
    Eh = E // 2
    lo32 = row[:Eh].astype(jnp.int32)
    hi32 = col[:Eh].astype(jnp.int32)

    cnt_lo_part, deg_part, key_pad = _sc_hist(lo32, hi32)
    cnt2, deg2, dsi2, fd2 = _tc_norm(cnt_lo_part, deg_part)
    cnt_lo = cnt2.reshape(-1)[:size]
    deg = deg2.reshape(-1)[:size]
    dsi = dsi2.reshape(-1)[:size]
    fd = fd2.reshape(-1)[:size]
    start_row = (jnp.cumsum(cnt_lo) - cnt_lo).astype(jnp.int32)

    k3, i3 = _sc_sort(key_pad)
    perm = i3[:Eh]
    skey = lax.bitcast_convert_type(k3[:Eh], jnp.uint32)
    t = jnp.arange(Eh, dtype=jnp.int32)
    is_new = jnp.concatenate([jnp.ones((1,), bool), skey[1:] != skey[:-1]])
    run_first = jax.lax.associative_scan(jnp.maximum, jnp.where(is_new, t, 0))
    mm_sorted = perm[run_first]
    mm = jnp.zeros((Eh,), jnp.int32).at[perm].set(mm_sorted, unique_indices=True)

    rev = jnp.concatenate([mm + Eh, mm]).astype(idt)
    full_left_right_idx = jnp.stack([jnp.arange(E, dtype=idt), rev])
    left_right_idx = jnp.stack([jnp.arange(Eh, dtype=idt), (mm + Eh).astype(idt)])
    vertex_tril_idx = edge_index[:, :Eh]

    dr = jnp.arange(final_d * size, dtype=idt)
    diag_indices = jnp.stack([dr, dr])
    diag_values = jnp.concatenate(
        [diag_maps, fd[:, None], fd[:, None]], axis=1).reshape(-1)

    r_s = lo32[perm]
    c_s = hi32[perm]
    start = start_row[r_s]
    cnt = cnt_lo[r_s]
    base = 3 * start + t
    w = dsi[r_s] * dsi[c_s]
    tm = tril_maps[perm]
    tv = jnp.zeros((4 * Eh,), jnp.float32)
    ti0 = jnp.zeros((4 * Eh,), jnp.int32)
    ti1 = jnp.zeros((4 * Eh,), jnp.int32)
    vals = [tm[:, 0], tm[:, 1], w, -w]
    for f in range(4):
        p = base + f * cnt
        tv = tv.at[p].set(vals[f], unique_indices=True)
        ti0 = ti0.at[p].set(4 * r_s + f, unique_indices=True)
        ti1 = ti1.at[p].set(4 * c_s + f, unique_indices=True)
    tril_indices = jnp.stack([ti0, ti1]).astype(idt)
    return (diag_indices, diag_values, tril_indices, tv, deg,
            full_left_right_idx, left_right_idx, vertex_tril_idx)


# speedup vs baseline: 2.3636x; 2.3636x over previous
import jax
import jax.numpy as jnp
from jax import lax
from jax.experimental import pallas as pl
from jax.experimental.pallas import tpu as pltpu
from jax.experimental.pallas import tpu_sc as plsc

_SIZE = 50000
_EH = 800000
_FINAL_D = 4
_PAD = 50176

_NC = 2
_NS = 16
_NW = _NC * _NS
_IDXB = 128
_NBLKS = _EH // _IDXB
_ZPT = _PAD // _NS

_NPAD = 802816
_SPT = _NPAD // _NW
_SKB = 512
_SNB = _SPT // _SKB
_PASSES = ((0, 0x7FF, 2048), (11, 0x7FF, 2048), (22, 0x3FF, 1024))


def _i32(x):
    return jnp.int32(x)


def _iota16():
    return lax.iota(jnp.int32, 16)


_GDN = lax.GatherDimensionNumbers(
    offset_dims=(), collapsed_slice_dims=(0,), start_index_map=(0,))


def _take(x, idx):
    return lax.gather(x, idx[:, None], _GDN, (1,),
                      mode=lax.GatherScatterMode.PROMISE_IN_BOUNDS)


def _count_body_factory(shift, mask, ndig):
    def body(key_hbm, cnt_hbm, keybuf, hist16, cntv):
        cid = lax.axis_index("c")
        sid = lax.axis_index("s")
        wid = cid * _NS + sid
        iota = _iota16()
        z16 = jnp.zeros((16,), jnp.int32)
        one16 = jnp.ones((16,), jnp.int32)

        def zero(i, c):
            hist16[pl.ds(i * 16, 16)] = z16
            return c

        lax.fori_loop(_i32(0), _i32(ndig), zero, _i32(0))
        base = wid * _SPT

        def blk(b, c):
            off = base + b * _SKB
            pltpu.sync_copy(key_hbm.at[pl.ds(off, _SKB)], keybuf)

            def vec(j, c2):
                k = keybuf[pl.ds(j * 16, 16)]
                d = lax.shift_right_logical(k, _i32(shift)) & _i32(mask)
                plsc.addupdate_scatter(hist16, [d * 16 + iota], one16)
                return c2

            lax.fori_loop(_i32(0), _i32(_SKB // 16), vec, _i32(0))
            return c

        lax.fori_loop(_i32(0), _i32(_SNB), blk, _i32(0))

        def red(i, c):
            acc = z16
            for l in range(16):
                acc = acc + plsc.load_gather(hist16, [i * 256 + iota * 16 + l])
            cntv[pl.ds(i * 16, 16)] = acc
            return c

        lax.fori_loop(_i32(0), _i32(ndig // 16), red, _i32(0))
        pltpu.sync_copy(cntv, cnt_hbm.at[wid])

    return body


def _sc_count(key_pad, shift, mask, ndig):
    f = pl.kernel(
        _count_body_factory(shift, mask, ndig),
        out_type=jax.ShapeDtypeStruct((_NW, ndig), jnp.int32),
        mesh=plsc.VectorSubcoreMesh(
            core_axis_name="c", subcore_axis_name="s",
            num_cores=_NC, num_subcores=_NS),
        compiler_params=pltpu.CompilerParams(needs_layout_passes=False),
        scratch_types=[
            pltpu.VMEM((_SKB,), jnp.int32),
            pltpu.VMEM((ndig * 16,), jnp.int32),
            pltpu.VMEM((ndig,), jnp.int32),
        ],
    )
    return f(key_pad)


def _permute_body_factory(shift, mask, ndig, has_idx):
    def body(*args):
        if has_idx:
            (key_hbm, idx_hbm, cnt_hbm, keyo, idxo,
             cntm, offt, keybuf, idxbuf, dpos, dkey, didx) = args
        else:
            (key_hbm, cnt_hbm, keyo, idxo,
             cntm, offt, keybuf, idxbuf, dpos, dkey, didx) = args
        cid = lax.axis_index("c")
        sid = lax.axis_index("s")
        wid = cid * _NS + sid
        iota = _iota16()
        z16 = jnp.zeros((16,), jnp.int32)
        pltpu.sync_copy(cnt_hbm, cntm)

        def chunk(i, carry):
            def tl(t, tp):
                tot, part = tp
                v = cntm[pl.ds(t * ndig + i * 16, 16)]
                tot = tot + v
                part = part + jnp.where(t < wid, v, z16)
                return (tot, part)

            tot, part = lax.fori_loop(_i32(0), _i32(_NW), tl, (z16, z16))
            incl = plsc.cumsum(tot)
            ex = incl - tot + carry
            offt[pl.ds(i * 16, 16)] = ex + part
            return carry + jnp.sum(tot, dtype=jnp.int32)

        lax.fori_loop(_i32(0), _i32(ndig // 16), chunk, _i32(0))

        base = wid * _SPT

        def blk(b, c):
            off0 = base + b * _SKB
            pltpu.sync_copy(key_hbm.at[pl.ds(off0, _SKB)], keybuf)
            if has_idx:
                pltpu.sync_copy(idx_hbm.at[pl.ds(off0, _SKB)], idxbuf)
            for j in range(_SKB // 16):
                k = keybuf[pl.ds(j * 16, 16)]
                if has_idx:
                    iv = idxbuf[pl.ds(j * 16, 16)]
                else:
                    iv = off0 + _i32(j * 16) + iota
                d = lax.shift_right_logical(k, _i32(shift)) & _i32(mask)
                comb = d * 16 + iota
                sk, sl = plsc.sort_key_val(comb, iota)
                ds_ = lax.shift_right_logical(sk, _i32(4))
                prev = _take(ds_, jnp.maximum(iota - 1, z16))
                isnew = (iota == 0) | (ds_ != prev)
                rs = plsc.cummax(jnp.where(isnew, iota, z16))
                w = iota - rs
                cur = plsc.load_gather(offt, [ds_])
                pos = cur + w
                nxt = _take(ds_, jnp.minimum(iota + 1, jnp.full((16,), 15, jnp.int32)))
                isend = (iota == 15) | (ds_ != nxt)
                plsc.addupdate_scatter(offt, [ds_], w + 1, mask=isend)
                ks = _take(k, sl)
                is_ = _take(iv, sl)
                r0, c0 = j // 8, j % 8
                dpos[r0, pl.ds(c0 * 16, 16)] = pos
                dkey[r0, pl.ds(c0 * 16, 16)] = ks
                didx[r0, pl.ds(c0 * 16, 16)] = is_
            for r in range(_SKB // 128):
                ri = _i32(r)
                pltpu.sync_copy(dkey.at[ri], keyo.at[dpos.at[ri]])
                pltpu.sync_copy(didx.at[ri], idxo.at[dpos.at[ri]])
            return c

        lax.fori_loop(_i32(0), _i32(_SNB), blk, _i32(0))

    return body


def _sc_permute(key_pad, idx_pad, cnt, shift, mask, ndig):
    has_idx = idx_pad is not None
    f = pl.kernel(
        _permute_body_factory(shift, mask, ndig, has_idx),
        out_type=(jax.ShapeDtypeStruct((_NPAD,), jnp.int32),
                  jax.ShapeDtypeStruct((_NPAD,), jnp.int32)),
        mesh=plsc.VectorSubcoreMesh(
            core_axis_name="c", subcore_axis_name="s",
            num_cores=_NC, num_subcores=_NS),
        compiler_params=pltpu.CompilerParams(needs_layout_passes=False),
        scratch_types=[
            pltpu.VMEM((_NW * ndig,), jnp.int32),
            pltpu.VMEM((ndig,), jnp.int32),
            pltpu.VMEM((_SKB,), jnp.int32),
            pltpu.VMEM((_SKB,), jnp.int32),
            pltpu.VMEM((_SKB // 128, 128), jnp.int32),
            pltpu.VMEM((_SKB // 128, 128), jnp.int32),
            pltpu.VMEM((_SKB // 128, 128), jnp.int32),
        ],
    )
    cnt1 = cnt.reshape(-1)
    if has_idx:
        return f(key_pad, idx_pad, cnt1)
    return f(key_pad, cnt1)


def _sc_sort(key_pad):
    k, i = key_pad, None
    for (shift, mask, ndig) in _PASSES:
        cnt = _sc_count(k, shift, mask, ndig)
        k, i = _sc_permute(k, i, cnt, shift, mask, ndig)
    return k, i


_N4 = 4 * _EH
_LR = 2 * _EH
_AKB = 896
_ANB = _SPT // _AKB


def _lane(x, lane):
    iota = _iota16()
    z16 = jnp.zeros((16,), jnp.int32)
    return jnp.sum(jnp.where(iota == lane, x, z16), dtype=jnp.int32)


def _meta_body(key_hbm, meta_hbm, keybuf, pbuf):
    cid = lax.axis_index("c")
    sid = lax.axis_index("s")
    wid = cid * _NS + sid
    iota = _iota16()
    z16 = jnp.zeros((16,), jnp.int32)
    neg16 = jnp.full((16,), -1, jnp.int32)
    base = wid * _SPT
    pbuf[pl.ds(0, 16)] = z16

    @pl.when(wid > 0)
    def _():
        pltpu.sync_copy(key_hbm.at[pl.ds(base - 16, 16)], pbuf)

    pk0 = _lane(pbuf[pl.ds(0, 16)], 15)

    def blk(b, carry):
        pk, mk, mr = carry
        off0 = base + b * _SKB
        pltpu.sync_copy(key_hbm.at[pl.ds(off0, _SKB)], keybuf)
        for j in range(_SKB // 16):
            k = keybuf[pl.ds(j * 16, 16)]
            posv = off0 + _i32(j * 16) + iota
            pv = jnp.where(iota == 0, jnp.broadcast_to(pk, (16,)),
                           _take(k, jnp.maximum(iota - 1, z16)))
            ink = (posv == 0) | (k != pv)
            r = lax.shift_right_logical(k, _i32(16))
            rp = lax.shift_right_logical(pv, _i32(16))
            inr = (posv == 0) | (r != rp)
            mk = jnp.maximum(mk, jnp.max(jnp.where(ink, posv, neg16)))
            mr = jnp.maximum(mr, jnp.max(jnp.where(inr, posv, neg16)))
            pk = _lane(k, 15)
        return (pk, mk, mr)

    pk, mk, mr = lax.fori_loop(
        _i32(0), _i32(_SNB), blk, (pk0, _i32(-1), _i32(-1)))
    outv = jnp.where(iota == 0, jnp.broadcast_to(mk, (16,)),
                     jnp.broadcast_to(mr, (16,)))
    pbuf[pl.ds(0, 16)] = outv
    pltpu.sync_copy(pbuf, meta_hbm.at[wid])


def _sc_meta(key_sorted):
    f = pl.kernel(
        _meta_body,
        out_type=jax.ShapeDtypeStruct((_NW, 16), jnp.int32),
        mesh=plsc.VectorSubcoreMesh(
            core_axis_name="c", subcore_axis_name="s",
            num_cores=_NC, num_subcores=_NS),
        compiler_params=pltpu.CompilerParams(needs_layout_passes=False),
        scratch_types=[
            pltpu.VMEM((_SKB,), jnp.int32),
            pltpu.VMEM((16,), jnp.int32),
        ],
    )
    return f(key_sorted)


def _asm_body(key_hbm, idx_hbm, meta_hbm, cnt_hbm, dsi_hbm, tm0_hbm, tm1_hbm,
              ti0_hbm, ti1_hbm, tvb_hbm, flr_hbm,
              keybuf, idxbuf, metab, pbuf,
              rfb, rsrb, rb, cb, pb,
              gmm, gcnt, gdr, gdc, gt0, gt1,
              si00, si01, si02, si03, si10, si11, si12, si13,
              sv0, sv1, sv2, sv3, sp0, sp1, sp2, sp3,
              lv1, lv2, lp1, lp2, gsem, ssem):
    cid = lax.axis_index("c")
    sid = lax.axis_index("s")
    wid = cid * _NS + sid
    iota = _iota16()
    z16 = jnp.zeros((16,), jnp.int32)
    base = wid * _SPT
    si0s = (si00, si01, si02, si03)
    si1s = (si10, si11, si12, si13)
    svs = (sv0, sv1, sv2, sv3)
    sps = (sp0, sp1, sp2, sp3)

    pltpu.sync_copy(meta_hbm, metab)
    pbuf[pl.ds(0, 16)] = z16

    @pl.when(wid > 0)
    def _():
        pltpu.sync_copy(key_hbm.at[pl.ds(base - 16, 16)], pbuf)

    pk0 = _lane(pbuf[pl.ds(0, 16)], 15)
    ck0 = _i32(-1)
    cr0 = _i32(-1)
    for tt in range(_NW):
        mv = metab[pl.ds(tt * 16, 16)]
        use = jnp.int32(tt) < wid
        ck0 = jnp.maximum(ck0, jnp.where(use, _lane(mv, 0), _i32(-1)))
        cr0 = jnp.maximum(cr0, jnp.where(use, _lane(mv, 1), _i32(-1)))

    def blk(b, carry):
        pk, ck, cr = carry
        off0 = base + b * _AKB
        pltpu.sync_copy(key_hbm.at[pl.ds(off0, _AKB)], keybuf)
        pltpu.sync_copy(idx_hbm.at[pl.ds(off0, _AKB)], idxbuf)
        for g in range(_AKB // 128):
            for j in range(8):
                o = g * 128 + j * 16
                k = keybuf[pl.ds(o, 16)]
                iv = idxbuf[pl.ds(o, 16)]
                posv = off0 + _i32(o) + iota
                pv = jnp.where(iota == 0, jnp.broadcast_to(pk, (16,)),
                               _take(k, jnp.maximum(iota - 1, z16)))
                ink = (posv == 0) | (k != pv)
                r = lax.shift_right_logical(k, _i32(16))
                rp = lax.shift_right_logical(pv, _i32(16))
                inr = (posv == 0) | (r != rp)
                rs_k = plsc.cummax(jnp.where(ink, posv, jnp.broadcast_to(ck, (16,))))
                rs_r = plsc.cummax(jnp.where(inr, posv, jnp.broadcast_to(cr, (16,))))
                ck = _lane(rs_k, 15)
                cr = _lane(rs_r, 15)
                pk = _lane(k, 15)
                c = k & _i32(0xFFFF)
                rfb[g, pl.ds(j * 16, 16)] = rs_k
                rsrb[g, pl.ds(j * 16, 16)] = rs_r
                rb[g, pl.ds(j * 16, 16)] = jnp.minimum(r, _i32(_PAD - 1))
                cb[g, pl.ds(j * 16, 16)] = jnp.minimum(c, _i32(_PAD - 1))
                pb[g, pl.ds(j * 16, 16)] = jnp.minimum(iv, _i32(_EH - 1))
        gsrcs = ((idx_hbm, rfb, gmm), (cnt_hbm, rb, gcnt), (dsi_hbm, rb, gdr),
                 (dsi_hbm, cb, gdc), (tm0_hbm, pb, gt0), (tm1_hbm, pb, gt1))
        descs = []
        for (src, ib, dst) in gsrcs:
            for ri in range(_AKB // 128):
                rii = _i32(ri)
                descs.append(
                    pltpu.async_copy(src.at[ib.at[rii]], dst.at[rii], gsem))
        for dd in descs:
            dd.wait()
        for g in range(_AKB // 128):
            for j in range(8):
                o = g * 128 + j * 16
                k = keybuf[pl.ds(o, 16)]
                posv = off0 + _i32(o) + iota
                real = posv < _i32(_EH)
                r = lax.shift_right_logical(k, _i32(16))
                c = k & _i32(0xFFFF)
                start = rsrb[g, pl.ds(j * 16, 16)]
                cntv = gcnt[g, pl.ds(j * 16, 16)]
                mm_s = gmm[g, pl.ds(j * 16, 16)]
                dr = gdr[g, pl.ds(j * 16, 16)]
                dc = gdc[g, pl.ds(j * 16, 16)]
                t0 = gt0[g, pl.ds(j * 16, 16)]
                t1 = gt1[g, pl.ds(j * 16, 16)]
                wv = dr * dc
                vb = (plsc.bitcast(t0, jnp.int32), plsc.bitcast(t1, jnp.int32),
                      plsc.bitcast(wv, jnp.int32), plsc.bitcast(-wv, jnp.int32))
                base3 = start * 3 + posv
                dump = _i32(_N4) + iota
                for fidx in range(4):
                    posf = base3 + _i32(fidx) * cntv
                    dest = jnp.where(real, posf, dump)
                    sps[fidx][g, pl.ds(j * 16, 16)] = dest
                    si0s[fidx][g, pl.ds(j * 16, 16)] = r * 4 + _i32(fidx)
                    si1s[fidx][g, pl.ds(j * 16, 16)] = c * 4 + _i32(fidx)
                    svs[fidx][g, pl.ds(j * 16, 16)] = vb[fidx]
                pbv = pb[g, pl.ds(j * 16, 16)]
                dumplr = _i32(_LR) + iota
                lv1[g, pl.ds(j * 16, 16)] = mm_s + _i32(_EH)
                lp1[g, pl.ds(j * 16, 16)] = jnp.where(real, pbv, dumplr)
                lv2[g, pl.ds(j * 16, 16)] = mm_s
                lp2[g, pl.ds(j * 16, 16)] = jnp.where(real, pbv + _i32(_EH), dumplr)
        descs2 = []
        for fidx in range(4):
            for ri in range(_AKB // 128):
                rii = _i32(ri)
                pidx = sps[fidx].at[rii]
                descs2.append(pltpu.async_copy(
                    si0s[fidx].at[rii], ti0_hbm.at[pidx], ssem))
                descs2.append(pltpu.async_copy(
                    si1s[fidx].at[rii], ti1_hbm.at[pidx], ssem))
                descs2.append(pltpu.async_copy(
                    svs[fidx].at[rii], tvb_hbm.at[pidx], ssem))
        for (vb_, pb_) in ((lv1, lp1), (lv2, lp2)):
            for ri in range(_AKB // 128):
                rii = _i32(ri)
                descs2.append(pltpu.async_copy(
                    vb_.at[rii], flr_hbm.at[pb_.at[rii]], ssem))
        for dd in descs2:
            dd.wait()
        return (pk, ck, cr)

    lax.fori_loop(_i32(0), _i32(_ANB), blk, (pk0, ck0, cr0))


def _sc_asm(key_sorted, idx_sorted, meta, cnt_lo_pad, dsi_pad, tm0, tm1):
    nr = _AKB // 128
    f = pl.kernel(
        _asm_body,
        out_type=(jax.ShapeDtypeStruct((_N4 + 16,), jnp.int32),
                  jax.ShapeDtypeStruct((_N4 + 16,), jnp.int32),
                  jax.ShapeDtypeStruct((_N4 + 16,), jnp.int32),
                  jax.ShapeDtypeStruct((_LR + 16,), jnp.int32)),
        mesh=plsc.VectorSubcoreMesh(
            core_axis_name="c", subcore_axis_name="s",
            num_cores=_NC, num_subcores=_NS),
        compiler_params=pltpu.CompilerParams(needs_layout_passes=False),
        scratch_types=[
            pltpu.VMEM((_AKB,), jnp.int32),
            pltpu.VMEM((_AKB,), jnp.int32),
            pltpu.VMEM((_NW * 16,), jnp.int32),
            pltpu.VMEM((16,), jnp.int32),
            pltpu.VMEM((nr, 128), jnp.int32),
            pltpu.VMEM((nr, 128), jnp.int32),
            pltpu.VMEM((nr, 128), jnp.int32),
            pltpu.VMEM((nr, 128), jnp.int32),
            pltpu.VMEM((nr, 128), jnp.int32),
            pltpu.VMEM((nr, 128), jnp.int32),
            pltpu.VMEM((nr, 128), jnp.int32),
            pltpu.VMEM((nr, 128), jnp.float32),
            pltpu.VMEM((nr, 128), jnp.float32),
            pltpu.VMEM((nr, 128), jnp.float32),
            pltpu.VMEM((nr, 128), jnp.float32),
            pltpu.VMEM((nr, 128), jnp.int32),
            pltpu.VMEM((nr, 128), jnp.int32),
            pltpu.VMEM((nr, 128), jnp.int32),
            pltpu.VMEM((nr, 128), jnp.int32),
            pltpu.VMEM((nr, 128), jnp.int32),
            pltpu.VMEM((nr, 128), jnp.int32),
            pltpu.VMEM((nr, 128), jnp.int32),
            pltpu.VMEM((nr, 128), jnp.int32),
            pltpu.VMEM((nr, 128), jnp.int32),
            pltpu.VMEM((nr, 128), jnp.int32),
            pltpu.VMEM((nr, 128), jnp.int32),
            pltpu.VMEM((nr, 128), jnp.int32),
            pltpu.VMEM((nr, 128), jnp.int32),
            pltpu.VMEM((nr, 128), jnp.int32),
            pltpu.VMEM((nr, 128), jnp.int32),
            pltpu.VMEM((nr, 128), jnp.int32),
            pltpu.VMEM((nr, 128), jnp.int32),
            pltpu.VMEM((nr, 128), jnp.int32),
            pltpu.VMEM((nr, 128), jnp.int32),
            pltpu.VMEM((nr, 128), jnp.int32),
            pltpu.SemaphoreType.DMA,
            pltpu.SemaphoreType.DMA,
        ],
    )
    return f(key_sorted, idx_sorted, meta.reshape(-1), cnt_lo_pad, dsi_pad,
             tm0, tm1)


def _iota_body(o_ref):
    o_ref[...] = (lax.broadcasted_iota(jnp.int32, (12544, 128), 0) * 128
                  + lax.broadcasted_iota(jnp.int32, (12544, 128), 1))


def _tc_iota():
    out = pl.pallas_call(
        _iota_body,
        out_shape=jax.ShapeDtypeStruct((12544, 128), jnp.int32),
    )()
    return out.reshape(-1)


def _hist_body(lo_hbm, hi_hbm, cnt_lo_hbm, deg_hbm, key_hbm,
               lo_v, hi_v, key_v, ones_v, zbuf_v, cnt_lo_s, deg_s):
    cid = lax.axis_index("c")
    sid = lax.axis_index("s")
    wid = cid * _NS + sid
    ones16 = jnp.ones((16,), jnp.int32)
    z16 = jnp.zeros((16,), jnp.int32)
    for j in range(_IDXB // 16):
        ones_v[pl.ds(j * 16, 16)] = ones16
    for j in range(_ZPT // 16):
        zbuf_v[pl.ds(j * 16, 16)] = z16
    pltpu.sync_copy(zbuf_v, cnt_lo_s.at[pl.ds(sid * _ZPT, _ZPT)])
    pltpu.sync_copy(zbuf_v, deg_s.at[pl.ds(sid * _ZPT, _ZPT)])
    plsc.subcore_barrier()

    nblk = jnp.int32(_NBLKS // _NW) + jnp.where(
        wid < _NBLKS % _NW, jnp.int32(1), jnp.int32(0))

    def blk(b, carry):
        off = (b * _NW + wid) * _IDXB
        pltpu.sync_copy(lo_hbm.at[pl.ds(off, _IDXB)], lo_v)
        pltpu.sync_copy(hi_hbm.at[pl.ds(off, _IDXB)], hi_v)
        for j in range(_IDXB // 16):
            l16 = lo_v[pl.ds(j * 16, 16)]
            h16 = hi_v[pl.ds(j * 16, 16)]
            key_v[pl.ds(j * 16, 16)] = lax.shift_left(l16, _i32(16)) | h16
        pltpu.sync_copy(key_v, key_hbm.at[pl.ds(off, _IDXB)])
        pltpu.sync_copy(ones_v, cnt_lo_s.at[lo_v], add=True)
        pltpu.sync_copy(ones_v, deg_s.at[lo_v], add=True)
        pltpu.sync_copy(ones_v, deg_s.at[hi_v], add=True)
        return carry

    lax.fori_loop(jnp.int32(0), nblk, blk, jnp.int32(0))

    @pl.when(wid == _NW - 1)
    def _():
        neg16 = jnp.full((16,), -1, jnp.int32)
        for j in range(_IDXB // 16):
            key_v[pl.ds(j * 16, 16)] = neg16

        def pblk(p, c):
            pltpu.sync_copy(key_v, key_hbm.at[pl.ds(_EH + p * _IDXB, _IDXB)])
            return c

        lax.fori_loop(_i32(0), _i32((_NPAD - _EH) // _IDXB), pblk, _i32(0))

    plsc.subcore_barrier()

    @pl.when(sid == 0)
    def _():
        pltpu.sync_copy(cnt_lo_s, cnt_lo_hbm.at[cid])
        pltpu.sync_copy(deg_s, deg_hbm.at[cid])


def _sc_hist(lo32, hi32):
    f = pl.kernel(
        _hist_body,
        out_type=(
            jax.ShapeDtypeStruct((_NC, _PAD), jnp.int32),
            jax.ShapeDtypeStruct((_NC, _PAD), jnp.int32),
            jax.ShapeDtypeStruct((_NPAD,), jnp.int32),
        ),
        mesh=plsc.VectorSubcoreMesh(
            core_axis_name="c", subcore_axis_name="s",
            num_cores=_NC, num_subcores=_NS),
        compiler_params=pltpu.CompilerParams(needs_layout_passes=False),
        scratch_types=[
            pltpu.VMEM((_IDXB,), jnp.int32),
            pltpu.VMEM((_IDXB,), jnp.int32),
            pltpu.VMEM((_IDXB,), jnp.int32),
            pltpu.VMEM((_IDXB,), jnp.int32),
            pltpu.VMEM((_ZPT,), jnp.int32),
            pltpu.VMEM_SHARED((_PAD,), jnp.int32),
            pltpu.VMEM_SHARED((_PAD,), jnp.int32),
        ],
    )
    return f(lo32, hi32)


def _norm_body(cnt_lo_p, deg_p, cnt_lo_ref, deg_ref, dsi_ref, fd_ref):
    cnt = cnt_lo_p[0] + cnt_lo_p[1]
    degi = deg_p[0] + deg_p[1]
    deg = degi.astype(jnp.float32)
    cnt_lo_ref[...] = cnt
    deg_ref[...] = deg
    dsi = lax.rsqrt(deg + 1.0)
    dsi_ref[...] = dsi
    fd_ref[...] = dsi * dsi * deg


def _tc_norm(cnt_lo_part, deg_part):
    return pl.pallas_call(
        _norm_body,
        out_shape=(jax.ShapeDtypeStruct((392, 128), jnp.int32),
                   jax.ShapeDtypeStruct((392, 128), jnp.float32),
                   jax.ShapeDtypeStruct((392, 128), jnp.float32),
                   jax.ShapeDtypeStruct((392, 128), jnp.float32)),
    )(cnt_lo_part.reshape(_NC, 392, 128), deg_part.reshape(_NC, 392, 128))


def kernel(edge_index, diag_maps, tril_maps):
    size, final_d = _SIZE, _FINAL_D
    idt = edge_index.dtype
    row, col = edge_index[0], edge_index[1]
    E = row.shape[0]
    Eh = E // 2
    lo32 = row[:Eh].astype(jnp.int32)
    hi32 = col[:Eh].astype(jnp.int32)

    cnt_lo_part, deg_part, key_pad = _sc_hist(lo32, hi32)
    cnt2, deg2, dsi2, fd2 = _tc_norm(cnt_lo_part, deg_part)
    deg = deg2.reshape(-1)[:size]
    fd = fd2.reshape(-1)[:size]

    k3, i3 = _sc_sort(key_pad)
    meta = _sc_meta(k3)
    ti0_b, ti1_b, tvb, flr_pad = _sc_asm(
        k3, i3, meta, cnt2.reshape(-1), dsi2.reshape(-1),
        tril_maps[:, 0], tril_maps[:, 1])
    ar = _tc_iota()

    flr = flr_pad[:_LR]
    full_left_right_idx = jnp.stack([ar[:E].astype(idt), flr.astype(idt)])
    left_right_idx = jnp.stack(
        [ar[:Eh].astype(idt), flr[:Eh].astype(idt)])
    vertex_tril_idx = edge_index[:, :Eh]

    dr = ar[:final_d * size].astype(idt)
    diag_indices = jnp.stack([dr, dr])
    diag_values = jnp.concatenate(
        [diag_maps, fd[:, None], fd[:, None]], axis=1).reshape(-1)

    tril_indices = jnp.stack(
        [ti0_b[:_N4].astype(idt), ti1_b[:_N4].astype(idt)])
    tv = lax.bitcast_convert_type(tvb[:_N4], jnp.float32)
    return (diag_indices, diag_values, tril_indices, tv, deg,
            full_left_right_idx, left_right_idx, vertex_tril_idx)
